# baseline (speedup 1.0000x reference)
import functools

import jax
import jax.numpy as jnp
from jax import lax
from jax.experimental import pallas as pl
from jax.experimental.pallas import tpu as pltpu
from jax.experimental.pallas import tpu_sc as plsc

N_ROWS = 100000
D = 128
G = 64
NC = 2
NS = 16
NW = NC * NS
MAIN = 3120
SUB = 120
NSUB = MAIN // SUB
EXTRA_BASE = NW * MAIN
N_EXTRA_W = (N_ROWS - EXTRA_BASE) // 8
IDS_PAD = 3152
CNT_W = 16


def _sc_pool_body(x_hbm, ids_hbm, part_hbm, cnt_hbm, xbuf, xbuf8,
                  ids_v, acc, cnt):
    cid = lax.axis_index("c")
    sid = lax.axis_index("s")
    wid = sid * NC + cid
    base = wid * MAIN

    pltpu.sync_copy(ids_hbm.at[pl.ds(base, MAIN)], ids_v.at[pl.ds(0, MAIN)])

    zeros = jnp.zeros((16,), jnp.float32)

    def zero_acc(i, carry):
        for cg in range(D // 16):
            acc[i, pl.ds(cg * 16, 16)] = zeros
        cnt[pl.ds(i * CNT_W, CNT_W)] = zeros
        return carry

    lax.fori_loop(0, G, zero_acc, 0)

    lanes = lax.iota(jnp.int32, 16)
    ones = jnp.ones((16,), jnp.float32)

    def count_body(b, carry):
        idsv = ids_v[pl.ds(b * 16, 16)]
        plsc.addupdate_scatter(cnt, [idsv * CNT_W + lanes], ones)
        return carry

    lax.fori_loop(0, MAIN // 16, count_body, 0)

    def sub_body(j, carry):
        pltpu.sync_copy(x_hbm.at[pl.ds(base + j * SUB, SUB)], xbuf)

        def row_body(r, c2):
            seg = ids_v[pl.ds(j * SUB + r, 16)][0]
            for cg in range(D // 16):
                plsc.addupdate(acc.at[seg, pl.ds(cg * 16, 16)],
                               xbuf[r, pl.ds(cg * 16, 16)])
            return c2

        return lax.fori_loop(0, SUB, row_body, carry)

    lax.fori_loop(0, NSUB, sub_body, 0)

    @pl.when(wid < N_EXTRA_W)
    def _extra():
        ebase = EXTRA_BASE + wid * 8
        pltpu.sync_copy(ids_hbm.at[pl.ds(ebase, 8)],
                        ids_v.at[pl.ds(MAIN, 8)])
        pltpu.sync_copy(x_hbm.at[pl.ds(ebase, 8)], xbuf8)

        def extra_row(r, c2):
            seg = ids_v[pl.ds(MAIN + r, 16)][0]
            for cg in range(D // 16):
                plsc.addupdate(acc.at[seg, pl.ds(cg * 16, 16)],
                               xbuf8[r, pl.ds(cg * 16, 16)])
            return c2

        lax.fori_loop(0, 8, extra_row, 0)
        idsv = ids_v[pl.ds(MAIN, 16)]
        plsc.addupdate_scatter(cnt, [idsv * CNT_W + lanes], ones,
                               mask=lanes < 8)

    pltpu.sync_copy(acc, part_hbm.at[wid])
    pltpu.sync_copy(cnt, cnt_hbm.at[wid])


_sc_pool = functools.partial(
    pl.kernel,
    out_type=[
        jax.ShapeDtypeStruct((NW, G, D), jnp.float32),
        jax.ShapeDtypeStruct((NW, G * CNT_W), jnp.float32),
    ],
    mesh=plsc.VectorSubcoreMesh(
        core_axis_name="c", subcore_axis_name="s", num_cores=NC,
        num_subcores=NS),
    compiler_params=pltpu.CompilerParams(needs_layout_passes=False),
    scratch_types=[
        pltpu.VMEM((SUB, D), jnp.float32),
        pltpu.VMEM((8, D), jnp.float32),
        pltpu.VMEM((IDS_PAD,), jnp.int32),
        pltpu.VMEM((G, D), jnp.float32),
        pltpu.VMEM((G * CNT_W,), jnp.float32),
    ],
)(_sc_pool_body)


def _finalize_body(part_ref, cnt_ref, o_ref):
    sums = jnp.sum(part_ref[...], axis=0)
    counts = jnp.sum(cnt_ref[...].reshape(NW, G, CNT_W), axis=(0, 2))
    o_ref[...] = sums / jnp.maximum(counts, 1.0)[:, None]


def kernel(x, pos, batch):
    del pos
    ids = batch.astype(jnp.int32)
    part, cnt = _sc_pool(x, ids)
    out = pl.pallas_call(
        _finalize_body,
        out_shape=jax.ShapeDtypeStruct((G, D), jnp.float32),
    )(part, cnt)
    return out


# speedup vs baseline: 1.9226x; 1.9226x over previous
import functools

import jax
import jax.numpy as jnp
from jax import lax
from jax.experimental import pallas as pl
from jax.experimental.pallas import tpu as pltpu
from jax.experimental.pallas import tpu_sc as plsc

N_ROWS = 100000
D = 128
G = 64
NC = 2
NS = 16
NW = NC * NS
MAIN = 3120
SUB = 120
NSUB = MAIN // SUB
EXTRA_BASE = NW * MAIN
N_EXTRA_W = (N_ROWS - EXTRA_BASE) // 8
IDS_PAD = 3152
CNT_W = 16


def _sc_pool_body(x_hbm, ids_hbm, part_hbm, cnt_hbm, xbuf, xbuf8,
                  ids_v, acc, cnt):
    cid = lax.axis_index("c")
    sid = lax.axis_index("s")
    wid = sid * NC + cid
    base = wid * MAIN

    pltpu.sync_copy(ids_hbm.at[pl.ds(base, MAIN)], ids_v.at[pl.ds(0, MAIN)])

    zeros = jnp.zeros((16,), jnp.float32)

    def zero_acc(i, carry):
        for cg in range(D // 16):
            acc[i, pl.ds(cg * 16, 16)] = zeros
        cnt[pl.ds(i * CNT_W, CNT_W)] = zeros
        return carry

    lax.fori_loop(0, G, zero_acc, 0)

    lanes = lax.iota(jnp.int32, 16)
    ones = jnp.ones((16,), jnp.float32)

    def count_body(b, carry):
        idsv = ids_v[pl.ds(b * 16, 16)]
        plsc.addupdate_scatter(cnt, [idsv * CNT_W + lanes], ones)
        return carry

    lax.fori_loop(0, MAIN // 16, count_body, 0)

    def flush(seg, accv):
        for cg in range(D // 16):
            plsc.addupdate(acc.at[seg, pl.ds(cg * 16, 16)], accv[cg])

    def sub_body(j, carry):
        pltpu.sync_copy(x_hbm.at[pl.ds(base + j * SUB, SUB)], xbuf)

        def grp_body(gi, c2):
            segv = ids_v[pl.ds(j * SUB + gi * 8, 16)]
            for jj in range(8):
                seg_prev = c2[0]
                accv = c2[1:]
                seg = segv[jj]
                change = seg != seg_prev

                @pl.when(change)
                def _(seg_prev=seg_prev, accv=accv):
                    flush(seg_prev, accv)

                keep = jnp.where(change, 0.0, 1.0)
                c2 = (seg,) + tuple(
                    accv[cg] * keep + xbuf[gi * 8 + jj, pl.ds(cg * 16, 16)]
                    for cg in range(D // 16))
            return c2

        return lax.fori_loop(0, SUB // 8, grp_body, carry)

    init = (ids_v[pl.ds(0, 16)][0],) + tuple(zeros for _ in range(D // 16))
    carry = lax.fori_loop(0, NSUB, sub_body, init)
    flush(carry[0], carry[1:])

    @pl.when(wid < N_EXTRA_W)
    def _extra():
        ebase = EXTRA_BASE + wid * 8
        pltpu.sync_copy(ids_hbm.at[pl.ds(ebase, 8)],
                        ids_v.at[pl.ds(MAIN, 8)])
        pltpu.sync_copy(x_hbm.at[pl.ds(ebase, 8)], xbuf8)

        def extra_row(r, c2):
            seg = ids_v[pl.ds(MAIN + r, 16)][0]
            for cg in range(D // 16):
                plsc.addupdate(acc.at[seg, pl.ds(cg * 16, 16)],
                               xbuf8[r, pl.ds(cg * 16, 16)])
            return c2

        lax.fori_loop(0, 8, extra_row, 0)
        idsv = ids_v[pl.ds(MAIN, 16)]
        plsc.addupdate_scatter(cnt, [idsv * CNT_W + lanes], ones,
                               mask=lanes < 8)

    pltpu.sync_copy(acc, part_hbm.at[wid])
    pltpu.sync_copy(cnt, cnt_hbm.at[wid])


_sc_pool = functools.partial(
    pl.kernel,
    out_type=[
        jax.ShapeDtypeStruct((NW, G, D), jnp.float32),
        jax.ShapeDtypeStruct((NW, G * CNT_W), jnp.float32),
    ],
    mesh=plsc.VectorSubcoreMesh(
        core_axis_name="c", subcore_axis_name="s", num_cores=NC,
        num_subcores=NS),
    compiler_params=pltpu.CompilerParams(needs_layout_passes=False),
    scratch_types=[
        pltpu.VMEM((SUB, D), jnp.float32),
        pltpu.VMEM((8, D), jnp.float32),
        pltpu.VMEM((IDS_PAD,), jnp.int32),
        pltpu.VMEM((G, D), jnp.float32),
        pltpu.VMEM((G * CNT_W,), jnp.float32),
    ],
)(_sc_pool_body)


def _finalize_body(part_ref, cnt_ref, o_ref):
    sums = jnp.sum(part_ref[...], axis=0)
    counts = jnp.sum(cnt_ref[...].reshape(NW, G, CNT_W), axis=(0, 2))
    o_ref[...] = sums / jnp.maximum(counts, 1.0)[:, None]


def kernel(x, pos, batch):
    del pos
    ids = batch.astype(jnp.int32)
    part, cnt = _sc_pool(x, ids)
    out = pl.pallas_call(
        _finalize_body,
        out_shape=jax.ShapeDtypeStruct((G, D), jnp.float32),
    )(part, cnt)
    return out


# speedup vs baseline: 2.0130x; 1.0470x over previous
import functools

import jax
import jax.numpy as jnp
from jax import lax
from jax.experimental import pallas as pl
from jax.experimental.pallas import tpu as pltpu
from jax.experimental.pallas import tpu_sc as plsc

N_ROWS = 100000
D = 128
G = 64
NC = 2
NS = 16
NW = NC * NS
MAIN = 3120
SUB = 240
NSUB = MAIN // SUB
EXTRA_BASE = NW * MAIN
N_EXTRA_W = (N_ROWS - EXTRA_BASE) // 8
IDS_PAD = 3152
CNT_W = 16


def _sc_pool_body(x_hbm, ids_hbm, part_hbm, cnt_hbm, xbuf, xbuf8,
                  ids_v, acc, cnt):
    cid = lax.axis_index("c")
    sid = lax.axis_index("s")
    wid = sid * NC + cid
    base = wid * MAIN

    pltpu.sync_copy(ids_hbm.at[pl.ds(base, MAIN)], ids_v.at[pl.ds(0, MAIN)])

    zeros = jnp.zeros((16,), jnp.float32)

    def zero_acc(i, carry):
        for cg in range(D // 16):
            acc[i, pl.ds(cg * 16, 16)] = zeros
        cnt[pl.ds(i * CNT_W, CNT_W)] = zeros
        return carry

    lax.fori_loop(0, G, zero_acc, 0)

    lanes = lax.iota(jnp.int32, 16)
    ones = jnp.ones((16,), jnp.float32)

    def count_body(b, carry):
        idsv = ids_v[pl.ds(b * 16, 16)]
        plsc.addupdate_scatter(cnt, [idsv * CNT_W + lanes], ones)
        return carry

    lax.fori_loop(0, MAIN // 16, count_body, 0)

    def sub_body(j, carry):
        pltpu.sync_copy(x_hbm.at[pl.ds(base + j * SUB, SUB)], xbuf)

        def grp_body(gi, c2):
            segv = ids_v[pl.ds(j * SUB + gi * 16, 16)]
            seg0 = segv[0]
            uniform = seg0 == segv[15]

            @pl.when(uniform)
            def _fast():
                for cg in range(D // 16):
                    s = xbuf[gi * 16, pl.ds(cg * 16, 16)]
                    for r in range(1, 16):
                        s = s + xbuf[gi * 16 + r, pl.ds(cg * 16, 16)]
                    plsc.addupdate(acc.at[seg0, pl.ds(cg * 16, 16)], s)

            @pl.when(jnp.logical_not(uniform))
            def _slow():
                for r in range(16):
                    seg = segv[r]
                    for cg in range(D // 16):
                        plsc.addupdate(acc.at[seg, pl.ds(cg * 16, 16)],
                                       xbuf[gi * 16 + r, pl.ds(cg * 16, 16)])

            return c2

        return lax.fori_loop(0, SUB // 16, grp_body, carry)

    lax.fori_loop(0, NSUB, sub_body, 0)

    @pl.when(wid < N_EXTRA_W)
    def _extra():
        ebase = EXTRA_BASE + wid * 8
        pltpu.sync_copy(ids_hbm.at[pl.ds(ebase, 8)],
                        ids_v.at[pl.ds(MAIN, 8)])
        pltpu.sync_copy(x_hbm.at[pl.ds(ebase, 8)], xbuf8)

        def extra_row(r, c2):
            seg = ids_v[pl.ds(MAIN + r, 16)][0]
            for cg in range(D // 16):
                plsc.addupdate(acc.at[seg, pl.ds(cg * 16, 16)],
                               xbuf8[r, pl.ds(cg * 16, 16)])
            return c2

        lax.fori_loop(0, 8, extra_row, 0)
        idsv = ids_v[pl.ds(MAIN, 16)]
        plsc.addupdate_scatter(cnt, [idsv * CNT_W + lanes], ones,
                               mask=lanes < 8)

    pltpu.sync_copy(acc, part_hbm.at[wid])
    pltpu.sync_copy(cnt, cnt_hbm.at[wid])


_sc_pool = functools.partial(
    pl.kernel,
    out_type=[
        jax.ShapeDtypeStruct((NW, G, D), jnp.float32),
        jax.ShapeDtypeStruct((NW, G * CNT_W), jnp.float32),
    ],
    mesh=plsc.VectorSubcoreMesh(
        core_axis_name="c", subcore_axis_name="s", num_cores=NC,
        num_subcores=NS),
    compiler_params=pltpu.CompilerParams(needs_layout_passes=False),
    scratch_types=[
        pltpu.VMEM((SUB, D), jnp.float32),
        pltpu.VMEM((8, D), jnp.float32),
        pltpu.VMEM((IDS_PAD,), jnp.int32),
        pltpu.VMEM((G, D), jnp.float32),
        pltpu.VMEM((G * CNT_W,), jnp.float32),
    ],
)(_sc_pool_body)


def _finalize_body(part_ref, cnt_ref, o_ref):
    sums = jnp.sum(part_ref[...], axis=0)
    counts = jnp.sum(cnt_ref[...].reshape(NW, G, CNT_W), axis=(0, 2))
    o_ref[...] = sums / jnp.maximum(counts, 1.0)[:, None]


def kernel(x, pos, batch):
    del pos
    ids = batch.astype(jnp.int32)
    part, cnt = _sc_pool(x, ids)
    out = pl.pallas_call(
        _finalize_body,
        out_shape=jax.ShapeDtypeStruct((G, D), jnp.float32),
    )(part, cnt)
    return out


# speedup vs baseline: 2.1373x; 1.0617x over previous
import functools

import jax
import jax.numpy as jnp
from jax import lax
from jax.experimental import pallas as pl
from jax.experimental.pallas import tpu as pltpu
from jax.experimental.pallas import tpu_sc as plsc

N_ROWS = 100000
D = 128
G = 64
NC = 2
NS = 16
NW = NC * NS
MAIN = 3120
SUB = 624
NSUB = MAIN // SUB
EXTRA_BASE = NW * MAIN
N_EXTRA_W = (N_ROWS - EXTRA_BASE) // 8
IDS_PAD = 3152
CNT_W = 16


def _sc_pool_body(x_hbm, ids_hbm, part_hbm, cnt_hbm, xbuf, xbuf8,
                  ids_v, acc, cnt):
    cid = lax.axis_index("c")
    sid = lax.axis_index("s")
    wid = sid * NC + cid
    base = wid * MAIN

    pltpu.sync_copy(ids_hbm.at[pl.ds(base, MAIN)], ids_v.at[pl.ds(0, MAIN)])

    zeros = jnp.zeros((16,), jnp.float32)

    def zero_acc(i, carry):
        for cg in range(D // 16):
            acc[i, pl.ds(cg * 16, 16)] = zeros
        cnt[pl.ds(i * CNT_W, CNT_W)] = zeros
        return carry

    lax.fori_loop(0, G, zero_acc, 0)

    lanes = lax.iota(jnp.int32, 16)
    ones = jnp.ones((16,), jnp.float32)

    def count_body(b, carry):
        idsv = ids_v[pl.ds(b * 16, 16)]
        plsc.addupdate_scatter(cnt, [idsv * CNT_W + lanes], ones)
        return carry

    lax.fori_loop(0, MAIN // 16, count_body, 0)

    def sub_body(j, carry):
        pltpu.sync_copy(x_hbm.at[pl.ds(base + j * SUB, SUB)], xbuf)

        def grp_body(gi, c2):
            segv = ids_v[pl.ds(j * SUB + gi * 16, 16)]
            seg0 = segv[0]
            uniform = seg0 == segv[15]

            @pl.when(uniform)
            def _fast():
                for cg in range(D // 16):
                    s = xbuf[gi * 16, pl.ds(cg * 16, 16)]
                    for r in range(1, 16):
                        s = s + xbuf[gi * 16 + r, pl.ds(cg * 16, 16)]
                    plsc.addupdate(acc.at[seg0, pl.ds(cg * 16, 16)], s)

            @pl.when(jnp.logical_not(uniform))
            def _slow():
                for r in range(16):
                    seg = segv[r]
                    for cg in range(D // 16):
                        plsc.addupdate(acc.at[seg, pl.ds(cg * 16, 16)],
                                       xbuf[gi * 16 + r, pl.ds(cg * 16, 16)])

            return c2

        return lax.fori_loop(0, SUB // 16, grp_body, carry)

    lax.fori_loop(0, NSUB, sub_body, 0)

    @pl.when(wid < N_EXTRA_W)
    def _extra():
        ebase = EXTRA_BASE + wid * 8
        pltpu.sync_copy(ids_hbm.at[pl.ds(ebase, 8)],
                        ids_v.at[pl.ds(MAIN, 8)])
        pltpu.sync_copy(x_hbm.at[pl.ds(ebase, 8)], xbuf8)

        def extra_row(r, c2):
            seg = ids_v[pl.ds(MAIN + r, 16)][0]
            for cg in range(D // 16):
                plsc.addupdate(acc.at[seg, pl.ds(cg * 16, 16)],
                               xbuf8[r, pl.ds(cg * 16, 16)])
            return c2

        lax.fori_loop(0, 8, extra_row, 0)
        idsv = ids_v[pl.ds(MAIN, 16)]
        plsc.addupdate_scatter(cnt, [idsv * CNT_W + lanes], ones,
                               mask=lanes < 8)

    pltpu.sync_copy(acc, part_hbm.at[wid])
    pltpu.sync_copy(cnt, cnt_hbm.at[wid])


_sc_pool = functools.partial(
    pl.kernel,
    out_type=[
        jax.ShapeDtypeStruct((NW, G, D), jnp.float32),
        jax.ShapeDtypeStruct((NW, G * CNT_W), jnp.float32),
    ],
    mesh=plsc.VectorSubcoreMesh(
        core_axis_name="c", subcore_axis_name="s", num_cores=NC,
        num_subcores=NS),
    compiler_params=pltpu.CompilerParams(needs_layout_passes=False),
    scratch_types=[
        pltpu.VMEM((SUB, D), jnp.float32),
        pltpu.VMEM((8, D), jnp.float32),
        pltpu.VMEM((IDS_PAD,), jnp.int32),
        pltpu.VMEM((G, D), jnp.float32),
        pltpu.VMEM((G * CNT_W,), jnp.float32),
    ],
)(_sc_pool_body)


def _finalize_body(part_ref, cnt_ref, o_ref):
    sums = jnp.sum(part_ref[...], axis=0)
    counts = jnp.sum(cnt_ref[...].reshape(NW, G, CNT_W), axis=(0, 2))
    o_ref[...] = sums / jnp.maximum(counts, 1.0)[:, None]


def kernel(x, pos, batch):
    del pos
    ids = batch.astype(jnp.int32)
    part, cnt = _sc_pool(x, ids)
    out = pl.pallas_call(
        _finalize_body,
        out_shape=jax.ShapeDtypeStruct((G, D), jnp.float32),
    )(part, cnt)
    return out


# speedup vs baseline: 2.4602x; 1.1511x over previous
import functools

import jax
import jax.numpy as jnp
from jax import lax
from jax.experimental import pallas as pl
from jax.experimental.pallas import tpu as pltpu
from jax.experimental.pallas import tpu_sc as plsc

N_ROWS = 100000
D = 128
G = 64
NC = 2
NS = 16
NW = NC * NS
MAIN = 3120
SUB = 240
NSUB = MAIN // SUB
EXTRA_BASE = NW * MAIN
N_EXTRA_W = (N_ROWS - EXTRA_BASE) // 8
IDS_PAD = 3152
CNT_W = 16


def _sc_pool_body(x_hbm, ids_hbm, part_hbm, cnt_hbm, xbuf, xbuf1, xbuf8,
                  ids_v, acc, cnt, sem0, sem1):
    cid = lax.axis_index("c")
    sid = lax.axis_index("s")
    wid = sid * NC + cid
    base = wid * MAIN

    pltpu.sync_copy(ids_hbm.at[pl.ds(base, MAIN)], ids_v.at[pl.ds(0, MAIN)])

    zeros = jnp.zeros((16,), jnp.float32)

    def zero_acc(i, carry):
        for cg in range(D // 16):
            acc[i, pl.ds(cg * 16, 16)] = zeros
        cnt[pl.ds(i * CNT_W, CNT_W)] = zeros
        return carry

    lax.fori_loop(0, G, zero_acc, 0)

    lanes = lax.iota(jnp.int32, 16)
    ones = jnp.ones((16,), jnp.float32)

    def count_body(b, carry):
        idsv = ids_v[pl.ds(b * 16, 16)]
        plsc.addupdate_scatter(cnt, [idsv * CNT_W + lanes], ones)
        return carry

    lax.fori_loop(0, MAIN // 16, count_body, 0)

    def process(buf, j):
        def grp_body(gi, c2):
            segv = ids_v[pl.ds(j * SUB + gi * 16, 16)]
            seg0 = segv[0]
            uniform = seg0 == segv[15]

            @pl.when(uniform)
            def _fast():
                for cg in range(D // 16):
                    s = buf[gi * 16, pl.ds(cg * 16, 16)]
                    for r in range(1, 16):
                        s = s + buf[gi * 16 + r, pl.ds(cg * 16, 16)]
                    plsc.addupdate(acc.at[seg0, pl.ds(cg * 16, 16)], s)

            @pl.when(jnp.logical_not(uniform))
            def _slow():
                for r in range(16):
                    seg = segv[r]
                    for cg in range(D // 16):
                        plsc.addupdate(acc.at[seg, pl.ds(cg * 16, 16)],
                                       buf[gi * 16 + r, pl.ds(cg * 16, 16)])

            return c2

        lax.fori_loop(0, SUB // 16, grp_body, 0)

    def start(j, buf, sem):
        pltpu.async_copy(x_hbm.at[pl.ds(base + j * SUB, SUB)], buf, sem)

    def wait(buf, sem):
        pltpu.make_async_copy(x_hbm.at[pl.ds(0, SUB)], buf, sem).wait()

    start(0, xbuf, sem0)
    start(1, xbuf1, sem1)

    def pair_body(p, carry):
        wait(xbuf, sem0)
        process(xbuf, 2 * p)
        start(2 * p + 2, xbuf, sem0)
        wait(xbuf1, sem1)
        process(xbuf1, 2 * p + 1)
        start(2 * p + 3, xbuf1, sem1)
        return carry

    lax.fori_loop(0, (NSUB - 3) // 2, pair_body, 0)
    wait(xbuf, sem0)
    process(xbuf, NSUB - 3)
    start(NSUB - 1, xbuf, sem0)
    wait(xbuf1, sem1)
    process(xbuf1, NSUB - 2)
    wait(xbuf, sem0)
    process(xbuf, NSUB - 1)

    @pl.when(wid < N_EXTRA_W)
    def _extra():
        ebase = EXTRA_BASE + wid * 8
        pltpu.sync_copy(ids_hbm.at[pl.ds(ebase, 8)],
                        ids_v.at[pl.ds(MAIN, 8)])
        pltpu.sync_copy(x_hbm.at[pl.ds(ebase, 8)], xbuf8)

        def extra_row(r, c2):
            seg = ids_v[pl.ds(MAIN + r, 16)][0]
            for cg in range(D // 16):
                plsc.addupdate(acc.at[seg, pl.ds(cg * 16, 16)],
                               xbuf8[r, pl.ds(cg * 16, 16)])
            return c2

        lax.fori_loop(0, 8, extra_row, 0)
        idsv = ids_v[pl.ds(MAIN, 16)]
        plsc.addupdate_scatter(cnt, [idsv * CNT_W + lanes], ones,
                               mask=lanes < 8)

    pltpu.sync_copy(acc, part_hbm.at[wid])
    pltpu.sync_copy(cnt, cnt_hbm.at[wid])


_sc_pool = functools.partial(
    pl.kernel,
    out_type=[
        jax.ShapeDtypeStruct((NW, G, D), jnp.float32),
        jax.ShapeDtypeStruct((NW, G * CNT_W), jnp.float32),
    ],
    mesh=plsc.VectorSubcoreMesh(
        core_axis_name="c", subcore_axis_name="s", num_cores=NC,
        num_subcores=NS),
    compiler_params=pltpu.CompilerParams(needs_layout_passes=False),
    scratch_types=[
        pltpu.VMEM((SUB, D), jnp.float32),
        pltpu.VMEM((SUB, D), jnp.float32),
        pltpu.VMEM((8, D), jnp.float32),
        pltpu.VMEM((IDS_PAD,), jnp.int32),
        pltpu.VMEM((G, D), jnp.float32),
        pltpu.VMEM((G * CNT_W,), jnp.float32),
        pltpu.SemaphoreType.DMA,
        pltpu.SemaphoreType.DMA,
    ],
)(_sc_pool_body)


def _finalize_body(part_ref, cnt_ref, o_ref):
    sums = jnp.sum(part_ref[...], axis=0)
    counts = jnp.sum(cnt_ref[...].reshape(NW, G, CNT_W), axis=(0, 2))
    o_ref[...] = sums / jnp.maximum(counts, 1.0)[:, None]


def kernel(x, pos, batch):
    del pos
    ids = batch.astype(jnp.int32)
    part, cnt = _sc_pool(x, ids)
    out = pl.pallas_call(
        _finalize_body,
        out_shape=jax.ShapeDtypeStruct((G, D), jnp.float32),
    )(part, cnt)
    return out


# speedup vs baseline: 3.1473x; 1.2793x over previous
import functools

import jax
import jax.numpy as jnp
from jax import lax
from jax.experimental import pallas as pl
from jax.experimental.pallas import tpu as pltpu
from jax.experimental.pallas import tpu_sc as plsc

N_ROWS = 100000
D = 128
G = 64
NC = 2
NS = 16
NW = NC * NS
MAIN = 3120
SUB = 240
NSUB = MAIN // SUB
EXTRA_BASE = NW * MAIN
N_EXTRA_W = (N_ROWS - EXTRA_BASE) // 8
IDS_PAD = 3152
CNT_W = 16


def _sc_pool_body(x_hbm, ids_hbm, part_hbm, cnt_hbm, xbuf, xbuf1, xbuf8,
                  ids_v, acc, cnt, sem0, sem1):
    cid = lax.axis_index("c")
    sid = lax.axis_index("s")
    wid = sid * NC + cid
    base = wid * MAIN

    pltpu.sync_copy(ids_hbm.at[pl.ds(base, MAIN)], ids_v.at[pl.ds(0, MAIN)])

    zeros = jnp.zeros((16,), jnp.float32)

    def zero_acc(i, carry):
        for cg in range(D // 16):
            acc[i, pl.ds(cg * 16, 16)] = zeros
        cnt[pl.ds(i * CNT_W, CNT_W)] = zeros
        return carry

    lax.fori_loop(0, G, zero_acc, 0)

    lanes = lax.iota(jnp.int32, 16)
    ones = jnp.ones((16,), jnp.float32)

    def count_body(b, carry):
        idsv = ids_v[pl.ds(b * 16, 16)]
        plsc.addupdate_scatter(cnt, [idsv * CNT_W + lanes], ones)
        return carry

    lax.fori_loop(0, MAIN // 16, count_body, 0)

    def process(buf, j):
        def grp_body(gi, c2):
            segv = ids_v[pl.ds(j * SUB + gi * 16, 16)]
            seg0 = segv[0]
            uniform = seg0 == segv[15]

            @pl.when(uniform)
            def _fast():
                s = [buf[gi * 16, pl.ds(cg * 16, 16)]
                     for cg in range(D // 16)]
                for r in range(1, 16):
                    for cg in range(D // 16):
                        s[cg] = s[cg] + buf[gi * 16 + r, pl.ds(cg * 16, 16)]
                for cg in range(D // 16):
                    plsc.addupdate(acc.at[seg0, pl.ds(cg * 16, 16)], s[cg])

            @pl.when(jnp.logical_not(uniform))
            def _slow():
                for r in range(16):
                    seg = segv[r]
                    for cg in range(D // 16):
                        plsc.addupdate(acc.at[seg, pl.ds(cg * 16, 16)],
                                       buf[gi * 16 + r, pl.ds(cg * 16, 16)])

            return c2

        lax.fori_loop(0, SUB // 16, grp_body, 0)

    def start(j, buf, sem):
        pltpu.async_copy(x_hbm.at[pl.ds(base + j * SUB, SUB)], buf, sem)

    def wait(buf, sem):
        pltpu.make_async_copy(x_hbm.at[pl.ds(0, SUB)], buf, sem).wait()

    start(0, xbuf, sem0)
    start(1, xbuf1, sem1)

    def pair_body(p, carry):
        wait(xbuf, sem0)
        process(xbuf, 2 * p)
        start(2 * p + 2, xbuf, sem0)
        wait(xbuf1, sem1)
        process(xbuf1, 2 * p + 1)
        start(2 * p + 3, xbuf1, sem1)
        return carry

    lax.fori_loop(0, (NSUB - 3) // 2, pair_body, 0)
    wait(xbuf, sem0)
    process(xbuf, NSUB - 3)
    start(NSUB - 1, xbuf, sem0)
    wait(xbuf1, sem1)
    process(xbuf1, NSUB - 2)
    wait(xbuf, sem0)
    process(xbuf, NSUB - 1)

    @pl.when(wid < N_EXTRA_W)
    def _extra():
        ebase = EXTRA_BASE + wid * 8
        pltpu.sync_copy(ids_hbm.at[pl.ds(ebase, 8)],
                        ids_v.at[pl.ds(MAIN, 8)])
        pltpu.sync_copy(x_hbm.at[pl.ds(ebase, 8)], xbuf8)

        def extra_row(r, c2):
            seg = ids_v[pl.ds(MAIN + r, 16)][0]
            for cg in range(D // 16):
                plsc.addupdate(acc.at[seg, pl.ds(cg * 16, 16)],
                               xbuf8[r, pl.ds(cg * 16, 16)])
            return c2

        lax.fori_loop(0, 8, extra_row, 0)
        idsv = ids_v[pl.ds(MAIN, 16)]
        plsc.addupdate_scatter(cnt, [idsv * CNT_W + lanes], ones,
                               mask=lanes < 8)

    pltpu.sync_copy(acc, part_hbm.at[wid])
    pltpu.sync_copy(cnt, cnt_hbm.at[wid])


_sc_pool = functools.partial(
    pl.kernel,
    out_type=[
        jax.ShapeDtypeStruct((NW, G, D), jnp.float32),
        jax.ShapeDtypeStruct((NW, G * CNT_W), jnp.float32),
    ],
    mesh=plsc.VectorSubcoreMesh(
        core_axis_name="c", subcore_axis_name="s", num_cores=NC,
        num_subcores=NS),
    compiler_params=pltpu.CompilerParams(needs_layout_passes=False),
    scratch_types=[
        pltpu.VMEM((SUB, D), jnp.float32),
        pltpu.VMEM((SUB, D), jnp.float32),
        pltpu.VMEM((8, D), jnp.float32),
        pltpu.VMEM((IDS_PAD,), jnp.int32),
        pltpu.VMEM((G, D), jnp.float32),
        pltpu.VMEM((G * CNT_W,), jnp.float32),
        pltpu.SemaphoreType.DMA,
        pltpu.SemaphoreType.DMA,
    ],
)(_sc_pool_body)


def _finalize_body(part_ref, cnt_ref, o_ref):
    sums = jnp.sum(part_ref[...], axis=0)
    counts = jnp.sum(cnt_ref[...].reshape(NW, G, CNT_W), axis=(0, 2))
    o_ref[...] = sums / jnp.maximum(counts, 1.0)[:, None]


def kernel(x, pos, batch):
    del pos
    ids = batch.astype(jnp.int32)
    part, cnt = _sc_pool(x, ids)
    out = pl.pallas_call(
        _finalize_body,
        out_shape=jax.ShapeDtypeStruct((G, D), jnp.float32),
    )(part, cnt)
    return out


# speedup vs baseline: 3.2385x; 1.0290x over previous
import functools

import jax
import jax.numpy as jnp
from jax import lax
from jax.experimental import pallas as pl
from jax.experimental.pallas import tpu as pltpu
from jax.experimental.pallas import tpu_sc as plsc

N_ROWS = 100000
D = 128
G = 64
NC = 2
NS = 16
NW = NC * NS
MAIN = 3120
SUB = 240
NSUB = MAIN // SUB
EXTRA_BASE = NW * MAIN
N_EXTRA_W = (N_ROWS - EXTRA_BASE) // 8
IDS_PAD = 3152
CNT_W = 16


def _sc_pool_body(x_hbm, ids_hbm, part_hbm, cnt_hbm, xbuf, xbuf1, xbuf8,
                  ids_v, acc, cnt, sem0, sem1):
    cid = lax.axis_index("c")
    sid = lax.axis_index("s")
    wid = sid * NC + cid
    base = wid * MAIN

    pltpu.sync_copy(ids_hbm.at[pl.ds(base, MAIN)], ids_v.at[pl.ds(0, MAIN)])

    zeros = jnp.zeros((16,), jnp.float32)

    def zero_acc(i, carry):
        for cg in range(D // 16):
            acc[i, pl.ds(cg * 16, 16)] = zeros
        cnt[pl.ds(i * CNT_W, CNT_W)] = zeros
        return carry

    lax.fori_loop(0, G, zero_acc, 0)

    lanes = lax.iota(jnp.int32, 16)
    ones = jnp.ones((16,), jnp.float32)

    def process(buf, j):
        def grp_body(gi, c2):
            segv = ids_v[pl.ds(j * SUB + gi * 16, 16)]
            seg0 = segv[0]
            uniform = seg0 == segv[15]

            @pl.when(uniform)
            def _fast():
                s = [buf[gi * 16, pl.ds(cg * 16, 16)]
                     for cg in range(D // 16)]
                for r in range(1, 16):
                    for cg in range(D // 16):
                        s[cg] = s[cg] + buf[gi * 16 + r, pl.ds(cg * 16, 16)]
                for cg in range(D // 16):
                    plsc.addupdate(acc.at[seg0, pl.ds(cg * 16, 16)], s[cg])
                plsc.addupdate(cnt.at[pl.ds(seg0 * CNT_W, CNT_W)], ones)

            @pl.when(jnp.logical_not(uniform))
            def _slow():
                plsc.addupdate_scatter(cnt, [segv * CNT_W + lanes], ones)
                for r in range(16):
                    seg = segv[r]
                    for cg in range(D // 16):
                        plsc.addupdate(acc.at[seg, pl.ds(cg * 16, 16)],
                                       buf[gi * 16 + r, pl.ds(cg * 16, 16)])

            return c2

        lax.fori_loop(0, SUB // 16, grp_body, 0)

    def start(j, buf, sem):
        pltpu.async_copy(x_hbm.at[pl.ds(base + j * SUB, SUB)], buf, sem)

    def wait(buf, sem):
        pltpu.make_async_copy(x_hbm.at[pl.ds(0, SUB)], buf, sem).wait()

    start(0, xbuf, sem0)
    start(1, xbuf1, sem1)

    def pair_body(p, carry):
        wait(xbuf, sem0)
        process(xbuf, 2 * p)
        start(2 * p + 2, xbuf, sem0)
        wait(xbuf1, sem1)
        process(xbuf1, 2 * p + 1)
        start(2 * p + 3, xbuf1, sem1)
        return carry

    lax.fori_loop(0, (NSUB - 3) // 2, pair_body, 0)
    wait(xbuf, sem0)
    process(xbuf, NSUB - 3)
    start(NSUB - 1, xbuf, sem0)
    wait(xbuf1, sem1)
    process(xbuf1, NSUB - 2)
    wait(xbuf, sem0)
    process(xbuf, NSUB - 1)

    @pl.when(wid < N_EXTRA_W)
    def _extra():
        ebase = EXTRA_BASE + wid * 8
        pltpu.sync_copy(ids_hbm.at[pl.ds(ebase, 8)],
                        ids_v.at[pl.ds(MAIN, 8)])
        pltpu.sync_copy(x_hbm.at[pl.ds(ebase, 8)], xbuf8)

        def extra_row(r, c2):
            seg = ids_v[pl.ds(MAIN + r, 16)][0]
            for cg in range(D // 16):
                plsc.addupdate(acc.at[seg, pl.ds(cg * 16, 16)],
                               xbuf8[r, pl.ds(cg * 16, 16)])
            return c2

        lax.fori_loop(0, 8, extra_row, 0)
        idsv = ids_v[pl.ds(MAIN, 16)]
        plsc.addupdate_scatter(cnt, [idsv * CNT_W + lanes], ones,
                               mask=lanes < 8)

    pltpu.sync_copy(acc, part_hbm.at[wid])
    pltpu.sync_copy(cnt, cnt_hbm.at[wid])


_sc_pool = functools.partial(
    pl.kernel,
    out_type=[
        jax.ShapeDtypeStruct((NW, G, D), jnp.float32),
        jax.ShapeDtypeStruct((NW, G * CNT_W), jnp.float32),
    ],
    mesh=plsc.VectorSubcoreMesh(
        core_axis_name="c", subcore_axis_name="s", num_cores=NC,
        num_subcores=NS),
    compiler_params=pltpu.CompilerParams(needs_layout_passes=False),
    scratch_types=[
        pltpu.VMEM((SUB, D), jnp.float32),
        pltpu.VMEM((SUB, D), jnp.float32),
        pltpu.VMEM((8, D), jnp.float32),
        pltpu.VMEM((IDS_PAD,), jnp.int32),
        pltpu.VMEM((G, D), jnp.float32),
        pltpu.VMEM((G * CNT_W,), jnp.float32),
        pltpu.SemaphoreType.DMA,
        pltpu.SemaphoreType.DMA,
    ],
)(_sc_pool_body)


def _finalize_body(part_ref, cnt_ref, o_ref):
    sums = jnp.sum(part_ref[...], axis=0)
    counts = jnp.sum(cnt_ref[...].reshape(NW, G, CNT_W), axis=(0, 2))
    o_ref[...] = sums / jnp.maximum(counts, 1.0)[:, None]


def kernel(x, pos, batch):
    del pos
    ids = batch.astype(jnp.int32)
    part, cnt = _sc_pool(x, ids)
    out = pl.pallas_call(
        _finalize_body,
        out_shape=jax.ShapeDtypeStruct((G, D), jnp.float32),
    )(part, cnt)
    return out
